# baseline (speedup 1.0000x reference)
import functools

import jax
import jax.numpy as jnp
from jax.experimental import pallas as pl
from jax.experimental.pallas import tpu as pltpu
from jax.experimental.pallas import tpu_sc as plsc

_N = 1024
_H = 256
_EPS_S = 0.005 ** 2
_SINK_ITERS = 20
_AUCTION_ITERS = 5
_BID_EPS = 1e-3
_PAD = 128
_NW = 32


def _assign_kernel(ct_ref, c_ref, n_ref, best_ref, x0p_ref):
    N = _N
    cloudT = ct_ref[0]
    cloud = c_ref[0]
    noise = n_ref[0]

    mu = jnp.mean(cloudT)
    std = jnp.sqrt(jnp.mean((cloudT - mu) ** 2))
    x0T = cloudT / std

    d2 = (noise[:, 0:1] - x0T[0:1, :]) ** 2
    d2 = d2 + (noise[:, 1:2] - x0T[1:2, :]) ** 2
    d2 = d2 + (noise[:, 2:3] - x0T[2:3, :]) ** 2

    C = d2 * 0.5
    loga = -jnp.log(jnp.float32(N))
    logb = -jnp.log(jnp.float32(N))

    log2e = jnp.float32(1.4426950408889634)
    Ceps2 = (C / _EPS_S) * log2e
    logb2 = logb * log2e
    loga2 = loga * log2e

    def sink(_, FG):
        F2, G2 = FG
        A = (G2 + logb2) - Ceps2
        m = jnp.max(A, axis=1, keepdims=True)
        F2 = -(jnp.log2(jnp.sum(jnp.exp2(A - m), axis=1, keepdims=True)) + m)
        A2 = (F2 + loga2) - Ceps2
        m2 = jnp.max(A2, axis=0, keepdims=True)
        G2 = -(jnp.log2(jnp.sum(jnp.exp2(A2 - m2), axis=0, keepdims=True)) + m2)
        return (F2, G2)

    f0 = jnp.zeros((N, 1), jnp.float32)
    g0 = jnp.zeros((1, N), jnp.float32)
    _, G2 = jax.lax.fori_loop(0, _SINK_ITERS, sink, (f0, g0))
    price = jnp.float32(_EPS_S * 0.6931471805599453) * (-G2)

    jcol = jax.lax.broadcasted_iota(jnp.int32, (N, N), 1).astype(jnp.float32)
    neg_inf = jnp.float32(-jnp.inf)
    bigf = jnp.float32(N)

    def auct(_, carry):
        price, _best = carry
        score = d2 + price
        s1 = jnp.max(score, axis=1, keepdims=True)
        idx = jnp.min(jnp.where(score == s1, jcol, bigf), axis=1,
                      keepdims=True)
        hit = jcol == idx
        s2 = jnp.max(jnp.where(hit, neg_inf, score), axis=1, keepdims=True)
        bid = (s2 - s1) + jnp.float32(_BID_EPS)
        scat = jnp.max(jnp.where(hit, bid, neg_inf), axis=0, keepdims=True)
        price = jnp.where(scat != neg_inf, scat, price)
        return (price, idx)

    best0 = jnp.zeros((N, 1), jnp.float32)
    price, _ = jax.lax.fori_loop(0, _AUCTION_ITERS - 1, auct, (price, best0))
    score = d2 + price
    s1 = jnp.max(score, axis=1, keepdims=True)
    best = jnp.min(jnp.where(score == s1, jcol, bigf), axis=1, keepdims=True)

    b = pl.program_id(0)
    best_ref[0] = jnp.swapaxes(best.astype(jnp.int32) + b * N, 0, 1)

    x0_rows = cloud / std
    x0p_ref[0] = jnp.concatenate(
        [x0_rows, jnp.zeros((N, _PAD - 3), jnp.float32)], axis=1)


def _sc_gather(table, idxflat):
    BN = table.shape[0]
    bpw = BN // _NW
    mesh = plsc.VectorSubcoreMesh(core_axis_name="c", subcore_axis_name="s")

    @functools.partial(
        pl.kernel, mesh=mesh,
        out_type=jax.ShapeDtypeStruct((BN, _PAD), jnp.float32),
        scratch_types=[
            pltpu.VMEM((bpw,), jnp.int32),
            pltpu.VMEM((bpw, _PAD), jnp.float32),
            pltpu.SemaphoreType.DMA,
        ],
    )
    def k(table_hbm, idx_hbm, out_hbm, idx_v, rows_v, sem):
        wid = jax.lax.axis_index("s") * 2 + jax.lax.axis_index("c")
        base = wid * bpw
        pltpu.sync_copy(idx_hbm.at[pl.ds(base, bpw)], idx_v)
        pltpu.async_copy(table_hbm.at[idx_v], rows_v, sem).wait()
        pltpu.sync_copy(rows_v, out_hbm.at[pl.ds(base, bpw)])

    return k(table, idxflat)


def _mlp_kernel(xal_ref, n_ref, t_ref, w1_ref, b1_ref, w2t_ref, b2_ref,
                vp_ref, v_ref):
    x0_al = xal_ref[0][:, 0:3]
    noise = n_ref[0]
    t = t_ref[0, 0, 0]

    x_t = (1.0 - t) * x0_al + t * noise
    v = noise - x0_al

    w1 = w1_ref[...]
    b1 = b1_ref[...]
    w2t = w2t_ref[...]
    b2 = b2_ref[...]
    pre = (x_t[:, 0:1] * w1[0:1, :] + x_t[:, 1:2] * w1[1:2, :]
           + x_t[:, 2:3] * w1[2:3, :] + t * w1[3:4, :] + b1)
    h = jnp.tanh(pre)
    vp = jax.lax.dot_general(h, w2t, (((1,), (1,)), ((), ())),
                             precision=jax.lax.Precision.HIGHEST,
                             preferred_element_type=jnp.float32) + b2

    vp_ref[0] = vp
    v_ref[0] = v


def kernel(cloud, noise, t, W1, b1, W2, b2):
    B = cloud.shape[0]
    nsplit = 2
    step = B // nsplit
    outs = [
        _half_pipeline(cloud[i:i + step], noise[i:i + step], t[i:i + step],
                       W1, b1, W2, b2)
        for i in range(0, B, step)
    ]
    vp = jnp.concatenate([o[0] for o in outs], axis=0)
    v = jnp.concatenate([o[1] for o in outs], axis=0)
    return (vp, v)


def _half_pipeline(cloud, noise, t, W1, b1, W2, b2):
    B, N, _ = cloud.shape
    H = W1.shape[1]
    cloudT = jnp.swapaxes(cloud, 1, 2)
    t3 = t.reshape(B, 1, 1)
    b1r = b1.reshape(1, H)
    W2T = W2.T
    b2r = b2.reshape(1, 3)

    best, x0p = pl.pallas_call(
        _assign_kernel,
        grid=(B,),
        in_specs=[
            pl.BlockSpec((1, 3, N), lambda b: (b, 0, 0)),
            pl.BlockSpec((1, N, 3), lambda b: (b, 0, 0)),
            pl.BlockSpec((1, N, 3), lambda b: (b, 0, 0)),
        ],
        out_specs=[
            pl.BlockSpec((1, 1, N), lambda b: (b, 0, 0)),
            pl.BlockSpec((1, N, _PAD), lambda b: (b, 0, 0)),
        ],
        out_shape=[
            jax.ShapeDtypeStruct((B, 1, N), jnp.int32),
            jax.ShapeDtypeStruct((B, N, _PAD), jnp.float32),
        ],
        compiler_params=pltpu.CompilerParams(
            dimension_semantics=("parallel",)),
    )(cloudT, cloud, noise)

    gathered = _sc_gather(x0p.reshape(B * N, _PAD), best.reshape(B * N))
    x0al = gathered.reshape(B, N, _PAD)

    vp, v = pl.pallas_call(
        _mlp_kernel,
        grid=(B,),
        in_specs=[
            pl.BlockSpec((1, N, _PAD), lambda b: (b, 0, 0)),
            pl.BlockSpec((1, N, 3), lambda b: (b, 0, 0)),
            pl.BlockSpec((1, 1, 1), lambda b: (b, 0, 0)),
            pl.BlockSpec((4, H), lambda b: (0, 0)),
            pl.BlockSpec((1, H), lambda b: (0, 0)),
            pl.BlockSpec((3, H), lambda b: (0, 0)),
            pl.BlockSpec((1, 3), lambda b: (0, 0)),
        ],
        out_specs=[
            pl.BlockSpec((1, N, 3), lambda b: (b, 0, 0)),
            pl.BlockSpec((1, N, 3), lambda b: (b, 0, 0)),
        ],
        out_shape=[jax.ShapeDtypeStruct((B, N, 3), jnp.float32)] * 2,
        compiler_params=pltpu.CompilerParams(
            dimension_semantics=("parallel",)),
    )(x0al, noise, t3, W1, b1r, W2T, b2r)
    return (vp, v)


# speedup vs baseline: 1.0032x; 1.0032x over previous
import functools

import jax
import jax.numpy as jnp
from jax.experimental import pallas as pl
from jax.experimental.pallas import tpu as pltpu
from jax.experimental.pallas import tpu_sc as plsc

_N = 1024
_H = 256
_EPS_S = 0.005 ** 2
_SINK_ITERS = 20
_AUCTION_ITERS = 5
_BID_EPS = 1e-3
_PAD = 128
_NW = 32


def _assign_kernel(ct_ref, c_ref, n_ref, best_ref, x0p_ref):
    N = _N
    cloudT = ct_ref[0]
    cloud = c_ref[0]
    noise = n_ref[0]

    mu = jnp.mean(cloudT)
    std = jnp.sqrt(jnp.mean((cloudT - mu) ** 2))
    x0T = cloudT / std

    d2 = (noise[:, 0:1] - x0T[0:1, :]) ** 2
    d2 = d2 + (noise[:, 1:2] - x0T[1:2, :]) ** 2
    d2 = d2 + (noise[:, 2:3] - x0T[2:3, :]) ** 2

    loga = -jnp.log(jnp.float32(N))
    logb = -jnp.log(jnp.float32(N))

    log2e = jnp.float32(1.4426950408889634)
    Ceps2 = ((d2 * 0.5) / _EPS_S) * log2e
    logb2 = logb * log2e
    loga2 = loga * log2e

    def sink(_, FG):
        F2, G2 = FG
        A = (G2 + logb2) - Ceps2
        m = jnp.max(A, axis=1, keepdims=True)
        F2 = -(jnp.log2(jnp.sum(jnp.exp2(A - m), axis=1, keepdims=True)) + m)
        A2 = (F2 + loga2) - Ceps2
        m2 = jnp.max(A2, axis=0, keepdims=True)
        G2 = -(jnp.log2(jnp.sum(jnp.exp2(A2 - m2), axis=0, keepdims=True)) + m2)
        return (F2, G2)

    f0 = jnp.zeros((N, 1), jnp.float32)
    g0 = jnp.zeros((1, N), jnp.float32)
    _, G2 = jax.lax.fori_loop(0, _SINK_ITERS, sink, (f0, g0))
    price = jnp.float32(_EPS_S * 0.6931471805599453) * (-G2)

    jcol = jax.lax.broadcasted_iota(jnp.int32, (N, N), 1).astype(jnp.float32)
    neg_inf = jnp.float32(-jnp.inf)
    bigf = jnp.float32(N)

    def auct(_, carry):
        price, _best = carry
        score = d2 + price
        s1 = jnp.max(score, axis=1, keepdims=True)
        idx = jnp.min(jnp.where(score == s1, jcol, bigf), axis=1,
                      keepdims=True)
        hit = jcol == idx
        s2 = jnp.max(jnp.where(hit, neg_inf, score), axis=1, keepdims=True)
        bid = (s2 - s1) + jnp.float32(_BID_EPS)
        scat = jnp.max(jnp.where(hit, bid, neg_inf), axis=0, keepdims=True)
        price = jnp.where(scat != neg_inf, scat, price)
        return (price, idx)

    best0 = jnp.zeros((N, 1), jnp.float32)
    price, _ = jax.lax.fori_loop(0, _AUCTION_ITERS - 1, auct, (price, best0))
    score = d2 + price
    s1 = jnp.max(score, axis=1, keepdims=True)
    best = jnp.min(jnp.where(score == s1, jcol, bigf), axis=1, keepdims=True)

    b = pl.program_id(0)
    best_ref[0] = jnp.swapaxes(best.astype(jnp.int32) + b * N, 0, 1)

    x0_rows = cloud / std
    x0p_ref[0] = jnp.concatenate(
        [x0_rows, jnp.zeros((N, _PAD - 3), jnp.float32)], axis=1)


def _sc_gather(table, idxflat):
    BN = table.shape[0]
    bpw = BN // _NW
    mesh = plsc.VectorSubcoreMesh(core_axis_name="c", subcore_axis_name="s")

    @functools.partial(
        pl.kernel, mesh=mesh,
        out_type=jax.ShapeDtypeStruct((BN, _PAD), jnp.float32),
        scratch_types=[
            pltpu.VMEM((bpw,), jnp.int32),
            pltpu.VMEM((bpw, _PAD), jnp.float32),
            pltpu.SemaphoreType.DMA,
        ],
    )
    def k(table_hbm, idx_hbm, out_hbm, idx_v, rows_v, sem):
        wid = jax.lax.axis_index("s") * 2 + jax.lax.axis_index("c")
        base = wid * bpw
        pltpu.sync_copy(idx_hbm.at[pl.ds(base, bpw)], idx_v)
        pltpu.async_copy(table_hbm.at[idx_v], rows_v, sem).wait()
        pltpu.sync_copy(rows_v, out_hbm.at[pl.ds(base, bpw)])

    return k(table, idxflat)


def _mlp_kernel(xal_ref, n_ref, t_ref, w1_ref, b1_ref, w2t_ref, b2_ref,
                vp_ref, v_ref):
    x0_al = xal_ref[0][:, 0:3]
    noise = n_ref[0]
    t = t_ref[0, 0, 0]

    x_t = (1.0 - t) * x0_al + t * noise
    v = noise - x0_al

    w1 = w1_ref[...]
    b1 = b1_ref[...]
    w2t = w2t_ref[...]
    b2 = b2_ref[...]
    pre = (x_t[:, 0:1] * w1[0:1, :] + x_t[:, 1:2] * w1[1:2, :]
           + x_t[:, 2:3] * w1[2:3, :] + t * w1[3:4, :] + b1)
    h = jnp.tanh(pre)
    vp = jnp.concatenate(
        [jnp.sum(h * w2t[c:c + 1, :], axis=1, keepdims=True) for c in range(3)],
        axis=1) + b2

    vp_ref[0] = vp
    v_ref[0] = v


def kernel(cloud, noise, t, W1, b1, W2, b2):
    B = cloud.shape[0]
    nsplit = 2
    step = B // nsplit
    outs = [
        _half_pipeline(cloud[i:i + step], noise[i:i + step], t[i:i + step],
                       W1, b1, W2, b2)
        for i in range(0, B, step)
    ]
    vp = jnp.concatenate([o[0] for o in outs], axis=0)
    v = jnp.concatenate([o[1] for o in outs], axis=0)
    return (vp, v)


def _half_pipeline(cloud, noise, t, W1, b1, W2, b2):
    B, N, _ = cloud.shape
    H = W1.shape[1]
    cloudT = jnp.swapaxes(cloud, 1, 2)
    t3 = t.reshape(B, 1, 1)
    b1r = b1.reshape(1, H)
    W2T = W2.T
    b2r = b2.reshape(1, 3)

    best, x0p = pl.pallas_call(
        _assign_kernel,
        grid=(B,),
        in_specs=[
            pl.BlockSpec((1, 3, N), lambda b: (b, 0, 0)),
            pl.BlockSpec((1, N, 3), lambda b: (b, 0, 0)),
            pl.BlockSpec((1, N, 3), lambda b: (b, 0, 0)),
        ],
        out_specs=[
            pl.BlockSpec((1, 1, N), lambda b: (b, 0, 0)),
            pl.BlockSpec((1, N, _PAD), lambda b: (b, 0, 0)),
        ],
        out_shape=[
            jax.ShapeDtypeStruct((B, 1, N), jnp.int32),
            jax.ShapeDtypeStruct((B, N, _PAD), jnp.float32),
        ],
        compiler_params=pltpu.CompilerParams(
            dimension_semantics=("parallel",)),
    )(cloudT, cloud, noise)

    gathered = _sc_gather(x0p.reshape(B * N, _PAD), best.reshape(B * N))
    x0al = gathered.reshape(B, N, _PAD)

    vp, v = pl.pallas_call(
        _mlp_kernel,
        grid=(B,),
        in_specs=[
            pl.BlockSpec((1, N, _PAD), lambda b: (b, 0, 0)),
            pl.BlockSpec((1, N, 3), lambda b: (b, 0, 0)),
            pl.BlockSpec((1, 1, 1), lambda b: (b, 0, 0)),
            pl.BlockSpec((4, H), lambda b: (0, 0)),
            pl.BlockSpec((1, H), lambda b: (0, 0)),
            pl.BlockSpec((3, H), lambda b: (0, 0)),
            pl.BlockSpec((1, 3), lambda b: (0, 0)),
        ],
        out_specs=[
            pl.BlockSpec((1, N, 3), lambda b: (b, 0, 0)),
            pl.BlockSpec((1, N, 3), lambda b: (b, 0, 0)),
        ],
        out_shape=[jax.ShapeDtypeStruct((B, N, 3), jnp.float32)] * 2,
        compiler_params=pltpu.CompilerParams(
            dimension_semantics=("parallel",)),
    )(x0al, noise, t3, W1, b1r, W2T, b2r)
    return (vp, v)


# speedup vs baseline: 1.0198x; 1.0165x over previous
import functools

import jax
import jax.numpy as jnp
from jax.experimental import pallas as pl
from jax.experimental.pallas import tpu as pltpu
from jax.experimental.pallas import tpu_sc as plsc

_N = 1024
_H = 256
_EPS_S = 0.005 ** 2
_SINK_ITERS = 20
_AUCTION_ITERS = 5
_BID_EPS = 1e-3
_PAD = 128
_NW = 32


def _assign_kernel(ct_ref, c_ref, n_ref, best_ref, x0p_ref):
    N = _N
    log2e = jnp.float32(1.4426950408889634)
    loga = -jnp.log(jnp.float32(N))
    logb = -jnp.log(jnp.float32(N))
    logb2 = logb * log2e
    loga2 = loga * log2e
    jcol = jax.lax.broadcasted_iota(jnp.int32, (N, N), 1).astype(jnp.float32)
    neg_inf = jnp.float32(-jnp.inf)
    bigf = jnp.float32(N)

    def prep(u):
        cloudT = ct_ref[u]
        noise = n_ref[u]
        mu = jnp.mean(cloudT)
        std = jnp.sqrt(jnp.mean((cloudT - mu) ** 2))
        x0T = cloudT / std
        d2 = (noise[:, 0:1] - x0T[0:1, :]) ** 2
        d2 = d2 + (noise[:, 1:2] - x0T[1:2, :]) ** 2
        d2 = d2 + (noise[:, 2:3] - x0T[2:3, :]) ** 2
        Ceps2 = ((d2 * 0.5) / _EPS_S) * log2e
        return d2, Ceps2, std

    d2a, Ca, stda = prep(0)
    d2b, Cb, stdb = prep(1)

    def half_sink(G2, Ceps2):
        A = (G2 + logb2) - Ceps2
        m = jnp.max(A, axis=1, keepdims=True)
        F2 = -(jnp.log2(jnp.sum(jnp.exp2(A - m), axis=1, keepdims=True)) + m)
        A2 = (F2 + loga2) - Ceps2
        m2 = jnp.max(A2, axis=0, keepdims=True)
        return -(jnp.log2(jnp.sum(jnp.exp2(A2 - m2), axis=0, keepdims=True))
                 + m2)

    def sink(_, GG):
        Ga, Gb = GG
        return (half_sink(Ga, Ca), half_sink(Gb, Cb))

    g0 = jnp.zeros((1, N), jnp.float32)
    Ga, Gb = jax.lax.fori_loop(0, _SINK_ITERS, sink, (g0, g0))
    psc = jnp.float32(_EPS_S * 0.6931471805599453)
    pricea = psc * (-Ga)
    priceb = psc * (-Gb)

    def half_auct(price, d2):
        score = d2 + price
        s1 = jnp.max(score, axis=1, keepdims=True)
        idx = jnp.min(jnp.where(score == s1, jcol, bigf), axis=1,
                      keepdims=True)
        hit = jcol == idx
        s2 = jnp.max(jnp.where(hit, neg_inf, score), axis=1, keepdims=True)
        bid = (s2 - s1) + jnp.float32(_BID_EPS)
        scat = jnp.max(jnp.where(hit, bid, neg_inf), axis=0, keepdims=True)
        return jnp.where(scat != neg_inf, scat, price), idx

    def auct(_, carry):
        pa, pb, _ba, _bb = carry
        pa, ba = half_auct(pa, d2a)
        pb, bb = half_auct(pb, d2b)
        return (pa, pb, ba, bb)

    best0 = jnp.zeros((N, 1), jnp.float32)
    pricea, priceb, _, _ = jax.lax.fori_loop(
        0, _AUCTION_ITERS - 1, auct, (pricea, priceb, best0, best0))

    def final_arg(price, d2):
        score = d2 + price
        s1 = jnp.max(score, axis=1, keepdims=True)
        return jnp.min(jnp.where(score == s1, jcol, bigf), axis=1,
                       keepdims=True)

    besta = final_arg(pricea, d2a)
    bestb = final_arg(priceb, d2b)

    b = pl.program_id(0)
    base = 2 * b * N
    best_ref[0] = jnp.swapaxes(besta.astype(jnp.int32) + base, 0, 1)
    best_ref[1] = jnp.swapaxes(bestb.astype(jnp.int32) + (base + N), 0, 1)

    zpad = jnp.zeros((N, _PAD - 3), jnp.float32)
    x0p_ref[0] = jnp.concatenate([c_ref[0] / stda, zpad], axis=1)
    x0p_ref[1] = jnp.concatenate([c_ref[1] / stdb, zpad], axis=1)


def _sc_gather(table, idxflat):
    BN = table.shape[0]
    bpw = BN // _NW
    mesh = plsc.VectorSubcoreMesh(core_axis_name="c", subcore_axis_name="s")

    @functools.partial(
        pl.kernel, mesh=mesh,
        out_type=jax.ShapeDtypeStruct((BN, _PAD), jnp.float32),
        scratch_types=[
            pltpu.VMEM((bpw,), jnp.int32),
            pltpu.VMEM((bpw, _PAD), jnp.float32),
            pltpu.SemaphoreType.DMA,
        ],
    )
    def k(table_hbm, idx_hbm, out_hbm, idx_v, rows_v, sem):
        wid = jax.lax.axis_index("s") * 2 + jax.lax.axis_index("c")
        base = wid * bpw
        pltpu.sync_copy(idx_hbm.at[pl.ds(base, bpw)], idx_v)
        pltpu.async_copy(table_hbm.at[idx_v], rows_v, sem).wait()
        pltpu.sync_copy(rows_v, out_hbm.at[pl.ds(base, bpw)])

    return k(table, idxflat)


def _mlp_kernel(xal_ref, n_ref, t_ref, w1_ref, b1_ref, w2t_ref, b2_ref,
                vp_ref, v_ref):
    x0_al = xal_ref[0][:, 0:3]
    noise = n_ref[0]
    t = t_ref[0, 0, 0]

    x_t = (1.0 - t) * x0_al + t * noise
    v = noise - x0_al

    w1 = w1_ref[...]
    b1 = b1_ref[...]
    w2t = w2t_ref[...]
    b2 = b2_ref[...]
    pre = (x_t[:, 0:1] * w1[0:1, :] + x_t[:, 1:2] * w1[1:2, :]
           + x_t[:, 2:3] * w1[2:3, :] + t * w1[3:4, :] + b1)
    h = jnp.tanh(pre)
    vp = jnp.concatenate(
        [jnp.sum(h * w2t[c:c + 1, :], axis=1, keepdims=True) for c in range(3)],
        axis=1) + b2

    vp_ref[0] = vp
    v_ref[0] = v


def kernel(cloud, noise, t, W1, b1, W2, b2):
    B = cloud.shape[0]
    nsplit = 2
    step = B // nsplit
    outs = [
        _half_pipeline(cloud[i:i + step], noise[i:i + step], t[i:i + step],
                       W1, b1, W2, b2)
        for i in range(0, B, step)
    ]
    vp = jnp.concatenate([o[0] for o in outs], axis=0)
    v = jnp.concatenate([o[1] for o in outs], axis=0)
    return (vp, v)


def _half_pipeline(cloud, noise, t, W1, b1, W2, b2):
    B, N, _ = cloud.shape
    H = W1.shape[1]
    cloudT = jnp.swapaxes(cloud, 1, 2)
    t3 = t.reshape(B, 1, 1)
    b1r = b1.reshape(1, H)
    W2T = W2.T
    b2r = b2.reshape(1, 3)

    best, x0p = pl.pallas_call(
        _assign_kernel,
        grid=(B // 2,),
        in_specs=[
            pl.BlockSpec((2, 3, N), lambda b: (b, 0, 0)),
            pl.BlockSpec((2, N, 3), lambda b: (b, 0, 0)),
            pl.BlockSpec((2, N, 3), lambda b: (b, 0, 0)),
        ],
        out_specs=[
            pl.BlockSpec((2, 1, N), lambda b: (b, 0, 0)),
            pl.BlockSpec((2, N, _PAD), lambda b: (b, 0, 0)),
        ],
        out_shape=[
            jax.ShapeDtypeStruct((B, 1, N), jnp.int32),
            jax.ShapeDtypeStruct((B, N, _PAD), jnp.float32),
        ],
        compiler_params=pltpu.CompilerParams(
            dimension_semantics=("parallel",)),
    )(cloudT, cloud, noise)

    gathered = _sc_gather(x0p.reshape(B * N, _PAD), best.reshape(B * N))
    x0al = gathered.reshape(B, N, _PAD)

    vp, v = pl.pallas_call(
        _mlp_kernel,
        grid=(B,),
        in_specs=[
            pl.BlockSpec((1, N, _PAD), lambda b: (b, 0, 0)),
            pl.BlockSpec((1, N, 3), lambda b: (b, 0, 0)),
            pl.BlockSpec((1, 1, 1), lambda b: (b, 0, 0)),
            pl.BlockSpec((4, H), lambda b: (0, 0)),
            pl.BlockSpec((1, H), lambda b: (0, 0)),
            pl.BlockSpec((3, H), lambda b: (0, 0)),
            pl.BlockSpec((1, 3), lambda b: (0, 0)),
        ],
        out_specs=[
            pl.BlockSpec((1, N, 3), lambda b: (b, 0, 0)),
            pl.BlockSpec((1, N, 3), lambda b: (b, 0, 0)),
        ],
        out_shape=[jax.ShapeDtypeStruct((B, N, 3), jnp.float32)] * 2,
        compiler_params=pltpu.CompilerParams(
            dimension_semantics=("parallel",)),
    )(x0al, noise, t3, W1, b1r, W2T, b2r)
    return (vp, v)
